# baseline (speedup 1.0000x reference)
import functools

import jax
import jax.numpy as jnp
from jax import lax
from jax.experimental import pallas as pl
from jax.experimental.pallas import tpu as pltpu, tpu_sc as plsc

_LANES = 16


def _round_up(x, m):
    return (x + m - 1) // m * m


@functools.partial(jax.jit, static_argnames=("n_anchors", "n_labels"))
def _sc_gather(b_idx, l_idx, x_idx, boxes_flat, scores_flat,
               n_anchors, n_labels):
    info = plsc.get_sparse_core_info()
    nw = info.num_cores * info.num_subcores
    chunk = b_idx.shape[0] // nw
    n_vec = chunk // _LANES
    out_len = b_idx.shape[0] * 7

    mesh = plsc.VectorSubcoreMesh(core_axis_name="c", subcore_axis_name="s")

    @functools.partial(
        pl.kernel,
        mesh=mesh,
        out_type=jax.ShapeDtypeStruct((out_len,), jnp.float32),
        compiler_params=pltpu.CompilerParams(needs_layout_passes=False),
        scratch_types=[
            pltpu.VMEM((chunk,), jnp.int32),
            pltpu.VMEM((chunk,), jnp.int32),
            pltpu.VMEM((chunk,), jnp.int32),
            pltpu.VMEM((chunk,), jnp.int32),
            pltpu.VMEM((chunk,), jnp.int32),
            pltpu.VMEM((chunk,), jnp.int32),
            pltpu.VMEM((chunk,), jnp.int32),
            pltpu.VMEM((chunk,), jnp.int32),
            pltpu.VMEM((chunk,), jnp.float32),
            pltpu.VMEM((chunk,), jnp.float32),
            pltpu.VMEM((chunk,), jnp.float32),
            pltpu.VMEM((chunk,), jnp.float32),
            pltpu.VMEM((chunk,), jnp.float32),
            pltpu.VMEM((chunk * 7,), jnp.float32),
            pltpu.SemaphoreType.DMA,
        ],
    )
    def body(b_hbm, l_hbm, x_hbm, boxes_hbm, scores_hbm, out_hbm,
             b_v, l_v, x_v, i0_v, i1_v, i2_v, i3_v, ie_v,
             g0_v, g1_v, g2_v, g3_v, gs_v, out_v, sem):
        wid = lax.axis_index("s") * info.num_cores + lax.axis_index("c")
        base = wid * chunk

        pltpu.sync_copy(b_hbm.at[pl.ds(base, chunk)], b_v)
        pltpu.sync_copy(l_hbm.at[pl.ds(base, chunk)], l_v)
        pltpu.sync_copy(x_hbm.at[pl.ds(base, chunk)], x_v)

        lane = lax.iota(jnp.int32, _LANES)

        def compute_indices(j, carry):
            off = j * _LANES
            b = b_v[pl.ds(off, _LANES)]
            lb = l_v[pl.ds(off, _LANES)]
            bx = x_v[pl.ds(off, _LANES)]
            row = b * n_anchors + bx
            r4 = row * 4
            i0_v[pl.ds(off, _LANES)] = r4
            i1_v[pl.ds(off, _LANES)] = r4 + 1
            i2_v[pl.ds(off, _LANES)] = r4 + 2
            i3_v[pl.ds(off, _LANES)] = r4 + 3
            ie_v[pl.ds(off, _LANES)] = row * n_labels + lb
            dst = (off + lane) * 7
            plsc.store_scatter(out_v, [dst], b.astype(jnp.float32))
            plsc.store_scatter(out_v, [dst + 6], lb.astype(jnp.float32))
            return carry

        lax.fori_loop(0, n_vec, compute_indices, 0)

        c0 = pltpu.async_copy(boxes_hbm.at[i0_v], g0_v, sem)
        c1 = pltpu.async_copy(boxes_hbm.at[i1_v], g1_v, sem)
        c2 = pltpu.async_copy(boxes_hbm.at[i2_v], g2_v, sem)
        c3 = pltpu.async_copy(boxes_hbm.at[i3_v], g3_v, sem)
        cs = pltpu.async_copy(scores_hbm.at[ie_v], gs_v, sem)
        c0.wait()
        c1.wait()
        c2.wait()
        c3.wait()
        cs.wait()

        def assemble(j, carry):
            off = j * _LANES
            dst = (off + lane) * 7
            plsc.store_scatter(out_v, [dst + 1], g0_v[pl.ds(off, _LANES)])
            plsc.store_scatter(out_v, [dst + 2], g1_v[pl.ds(off, _LANES)])
            plsc.store_scatter(out_v, [dst + 3], g2_v[pl.ds(off, _LANES)])
            plsc.store_scatter(out_v, [dst + 4], g3_v[pl.ds(off, _LANES)])
            plsc.store_scatter(out_v, [dst + 5], gs_v[pl.ds(off, _LANES)])
            return carry

        lax.fori_loop(0, n_vec, assemble, 0)

        pltpu.sync_copy(out_v, out_hbm.at[pl.ds(base * 7, chunk * 7)])

    return body(b_idx, l_idx, x_idx, boxes_flat, scores_flat)


def kernel(pred_boxes, pred_scores, selected_indexes):
    n_batch, n_anchors, n_box = pred_boxes.shape
    n_labels = pred_scores.shape[-1]
    n_rows = selected_indexes.shape[0]

    nw = 32
    padded = _round_up(n_rows, nw * _LANES)
    si = jnp.pad(selected_indexes, ((0, padded - n_rows), (0, 0)))
    b_idx = si[:, 0]
    l_idx = si[:, 1]
    x_idx = si[:, 2]

    boxes_flat = pred_boxes.reshape(-1)
    scores_flat = pred_scores.reshape(-1)

    out_flat = _sc_gather(b_idx, l_idx, x_idx, boxes_flat, scores_flat,
                          n_anchors, n_labels)
    return out_flat.reshape(padded, 7)[:n_rows]


# speedup vs baseline: 15.2698x; 15.2698x over previous
import functools

import jax
import jax.numpy as jnp
from jax import lax
from jax.experimental import pallas as pl
from jax.experimental.pallas import tpu as pltpu, tpu_sc as plsc

_LANES = 16
_W = 16


def _round_up(x, m):
    return (x + m - 1) // m * m


@functools.partial(jax.jit, static_argnames=())
def _sc_gather(b_idx, l_idx, x_idx, boxes_t, scores2d):
    info = plsc.get_sparse_core_info()
    nw = info.num_cores * info.num_subcores
    padded = b_idx.shape[0]
    chunk = padded // nw
    n_grp = chunk // _LANES
    n_batch = boxes_t.shape[0]

    mesh = plsc.VectorSubcoreMesh(core_axis_name="c", subcore_axis_name="s")

    @functools.partial(
        pl.kernel,
        mesh=mesh,
        out_type=jax.ShapeDtypeStruct((8, padded), jnp.float32),
        compiler_params=pltpu.CompilerParams(needs_layout_passes=False),
        scratch_types=[
            pltpu.VMEM((chunk,), jnp.int32),
            pltpu.VMEM((chunk,), jnp.int32),
            pltpu.VMEM((chunk,), jnp.int32),
            pltpu.VMEM((_LANES, 8, 128), jnp.float32),
            pltpu.VMEM((_LANES, 4, 128), jnp.float32),
            pltpu.VMEM((8, chunk), jnp.float32),
            pltpu.SemaphoreType.DMA,
        ],
    )
    def body(b_hbm, l_hbm, x_hbm, boxes_hbm, scores_hbm, out_hbm,
             b_v, l_v, x_v, sw_v, bw_v, out_v, sem):
        wid = lax.axis_index("s") * info.num_cores + lax.axis_index("c")
        base = wid * chunk

        pltpu.sync_copy(b_hbm.at[pl.ds(base, chunk)], b_v)
        pltpu.sync_copy(l_hbm.at[pl.ds(base, chunk)], l_v)
        pltpu.sync_copy(x_hbm.at[pl.ds(base, chunk)], x_v)

        lane16 = lax.iota(jnp.int32, _LANES)

        def group(g, carry):
            off = g * _LANES
            b16 = b_v[pl.ds(off, _LANES)]
            l16 = l_v[pl.ds(off, _LANES)]
            x16 = x_v[pl.ds(off, _LANES)]
            row16 = l16 * n_batch + b16
            rowa16 = lax.bitwise_and(row16, ~7)
            xa16 = lax.bitwise_and(x16, ~127)
            copies = []
            for k in range(_LANES):
                b = b16[k]
                rowa = pl.multiple_of(rowa16[k], 8)
                xa = pl.multiple_of(xa16[k], 128)
                copies.append(pltpu.async_copy(
                    scores_hbm.at[pl.ds(rowa, 8), pl.ds(xa, 128)],
                    sw_v.at[k], sem))
                copies.append(pltpu.async_copy(
                    boxes_hbm.at[b, :, pl.ds(xa, 128)], bw_v.at[k], sem))
            out_v[0, pl.ds(off, _LANES)] = b16.astype(jnp.float32)
            out_v[6, pl.ds(off, _LANES)] = l16.astype(jnp.float32)
            for c in copies:
                c.wait()
            col = lax.bitwise_and(x16, 127)
            subrow = lax.bitwise_and(row16, 7)
            out_v[5, pl.ds(off, _LANES)] = plsc.load_gather(
                sw_v, [lane16, subrow, col])
            for c in range(4):
                cc = jnp.full((_LANES,), c, jnp.int32)
                out_v[1 + c, pl.ds(off, _LANES)] = plsc.load_gather(
                    bw_v, [lane16, cc, col])
            return carry

        lax.fori_loop(0, n_grp, group, 0)

        pltpu.sync_copy(out_v, out_hbm.at[:, pl.ds(base, chunk)])

    return body(b_idx, l_idx, x_idx, boxes_t, scores2d)


def kernel(pred_boxes, pred_scores, selected_indexes):
    n_batch, n_anchors, n_box = pred_boxes.shape
    n_labels = pred_scores.shape[-1]
    n_rows = selected_indexes.shape[0]

    nw = 32
    padded = _round_up(n_rows, nw * _LANES)
    si = jnp.pad(selected_indexes, ((0, padded - n_rows), (0, 0)))
    b_idx = si[:, 0]
    l_idx = si[:, 1]
    x_idx = si[:, 2]

    scores2d = jnp.transpose(pred_scores, (2, 0, 1)).reshape(
        n_labels * n_batch, n_anchors)
    boxes_t = jnp.transpose(pred_boxes, (0, 2, 1))

    out_t = _sc_gather(b_idx, l_idx, x_idx, boxes_t, scores2d)
    return out_t[:7, :n_rows].T
